# Initial kernel scaffold; baseline (speedup 1.0000x reference)
import jax
import jax.numpy as jnp
from jax.experimental import pallas as pl


def kernel(pos, helix, W1a, b1a, g1, be1, W1b, b1b, W2a, b2a, g2, be2, W2b, b2b):
    raise NotImplementedError("write your pallas kernel here")


# speedup vs baseline: 1.1094x; 1.1094x over previous
import jax
import jax.numpy as jnp
from jax.experimental import pallas as pl
from jax.experimental.pallas import tpu as pltpu

_R2 = 6.25


def _bf(x):
    return x.astype(jnp.bfloat16).astype(jnp.float32)
_BI = 200
_BJ = 200


def _phase1_body(pos_i, pos_j, hx_is, hx_jc, w1a, b1a,
                 hcol, cnt, sumh, sumh2, ai_scr):
    jb = pl.program_id(0)
    ib = pl.program_id(1)

    pi = pos_i[...]
    pj = pos_j[...]
    bi = pi.shape[0]
    w1a_bf = _bf(w1a[...])
    pi_bf = _bf(pi)
    pj_bf = _bf(pj)
    ai_scr[...] = jnp.dot(pi_bf, w1a_bf[0:3, :],
                          preferred_element_type=jnp.float32) + b1a[...]
    bj = jnp.dot(pj_bf, w1a_bf[3:6, :],
                 preferred_element_type=jnp.float32)

    n2jc = jnp.sum(pj * pj, axis=1, keepdims=True)
    hxj = hx_jc[...]

    def body(i, carry):
        acc, acc2, ccnt = carry
        pirow = pos_i[pl.ds(i, 1), :]
        arow = ai_scr[pl.ds(i, 1), :]
        gcol = jax.lax.dot_general(pj_bf, _bf(pirow),
                                   (((1,), (1,)), ((), ())),
                                   preferred_element_type=jnp.float32)
        n2ii = jnp.sum(pirow * pirow, axis=1, keepdims=True)
        d2col = n2jc + n2ii - 2.0 * gcol
        mcol = ((d2col <= _R2) & (hxj != hx_is[0, 0, i])).astype(jnp.float32)
        pcol = (mcol - 1.0) * 1e30
        hmr = jnp.maximum(bj + arow + pcol, 0.0)
        return acc + hmr, acc2 + hmr * hmr, ccnt + mcol

    zero = jnp.zeros(bj.shape, jnp.float32)
    acc, acc2, ccnt = jax.lax.fori_loop(
        0, bi, body, (zero, zero, jnp.zeros((bj.shape[0], 1), jnp.float32)))

    @pl.when(ib == 0)
    def _():
        hcol[...] = jnp.zeros_like(hcol)
        cnt[...] = jnp.zeros_like(cnt)

    @pl.when((ib == 0) & (jb == 0))
    def _():
        sumh[...] = jnp.zeros_like(sumh)
        sumh2[...] = jnp.zeros_like(sumh2)

    hcol[...] += acc
    cnt[...] += ccnt
    sumh[...] += jnp.sum(acc, axis=0, keepdims=True)
    sumh2[...] += jnp.sum(acc2, axis=0, keepdims=True)


def _phase2_body(pos, hcol, cnt, sumh, sumh2, w1b, b1b, g1, be1,
                 w2a, b2a, g2, be2, w2b, b2b, out):
    e = jnp.sum(cnt[...])
    m = sumh[...] / e
    v = sumh2[...] / e - m * m
    scale = g1[...] * jax.lax.rsqrt(v + 1e-5)
    ceff = jnp.dot(be1[...] - m * scale, w1b[...],
                   preferred_element_type=jnp.float32) + b1b[...]
    s = jnp.dot(hcol[...] * scale, w1b[...],
                preferred_element_type=jnp.float32) + cnt[...] * ceff
    u = s / jnp.maximum(cnt[...], 1.0)

    w2a_bf = _bf(w2a[...])
    t = (jnp.dot(_bf(pos[...]), w2a_bf[0:3, :],
                 preferred_element_type=jnp.float32)
         + jnp.dot(_bf(u), w2a_bf[3:6, :],
                   preferred_element_type=jnp.float32)
         + b2a[...])
    t = jnp.maximum(t, 0.0)
    m2 = jnp.mean(t, axis=0, keepdims=True)
    d = t - m2
    v2 = jnp.mean(d * d, axis=0, keepdims=True)
    tn = d * jax.lax.rsqrt(v2 + 1e-5) * g2[...] + be2[...]
    out[...] = jnp.dot(_bf(tn), _bf(w2b[...]),
                       preferred_element_type=jnp.float32) + b2b[...]


def kernel(pos, helix, W1a, b1a, g1, be1, W1b, b1b, W2a, b2a, g2, be2, W2b, b2b):
    n = pos.shape[0]
    nj = n // _BJ
    ni = n // _BI
    hx_r = helix.reshape(ni, 1, _BI)
    hx_c = helix.reshape(n, 1)
    b1a2 = b1a.reshape(1, -1)

    hcol, cnt, sumh, sumh2 = pl.pallas_call(
        _phase1_body,
        grid=(nj, ni),
        in_specs=[
            pl.BlockSpec((_BI, 3), lambda jb, ib: (ib, 0)),
            pl.BlockSpec((_BJ, 3), lambda jb, ib: (jb, 0)),
            pl.BlockSpec((1, 1, _BI), lambda jb, ib: (ib, 0, 0),
                         memory_space=pltpu.SMEM),
            pl.BlockSpec((_BJ, 1), lambda jb, ib: (jb, 0)),
            pl.BlockSpec((6, 128), lambda jb, ib: (0, 0)),
            pl.BlockSpec((1, 128), lambda jb, ib: (0, 0)),
        ],
        out_specs=[
            pl.BlockSpec((_BJ, 128), lambda jb, ib: (jb, 0)),
            pl.BlockSpec((_BJ, 1), lambda jb, ib: (jb, 0)),
            pl.BlockSpec((1, 128), lambda jb, ib: (0, 0)),
            pl.BlockSpec((1, 128), lambda jb, ib: (0, 0)),
        ],
        out_shape=[
            jax.ShapeDtypeStruct((n, 128), jnp.float32),
            jax.ShapeDtypeStruct((n, 1), jnp.float32),
            jax.ShapeDtypeStruct((1, 128), jnp.float32),
            jax.ShapeDtypeStruct((1, 128), jnp.float32),
        ],
        scratch_shapes=[pltpu.VMEM((_BI, 128), jnp.float32)],
        compiler_params=pltpu.CompilerParams(
            dimension_semantics=("arbitrary", "arbitrary")),
    )(pos, pos, hx_r, hx_c, W1a, b1a2)

    out = pl.pallas_call(
        _phase2_body,
        out_shape=jax.ShapeDtypeStruct((n, 3), jnp.float32),
    )(pos, hcol, cnt, sumh, sumh2,
      W1b, b1b.reshape(1, -1), g1.reshape(1, -1), be1.reshape(1, -1),
      W2a, b2a.reshape(1, -1), g2.reshape(1, -1), be2.reshape(1, -1),
      W2b, b2b.reshape(1, -1))
    return out


# speedup vs baseline: 9.1626x; 8.2592x over previous
import jax
import jax.numpy as jnp
from jax.experimental import pallas as pl
from jax.experimental.pallas import tpu as pltpu

_R2 = 6.25


def _bf(x):
    return x.astype(jnp.bfloat16).astype(jnp.float32)
_BI = 200
_BJ = 200


def _phase1_body(pos_i, pos_j, hx_is, hx_jc, n2jr, w1a, b1a,
                 hcol, cnt, sumh, sumh2):
    jb = pl.program_id(0)
    ib = pl.program_id(1)

    pi = pos_i[...]
    pj = pos_j[...]
    bi = pi.shape[0]
    w1a_bf = _bf(w1a[...])
    pi_bf = _bf(pi)
    pj_bf = _bf(pj)
    ai = jnp.dot(pi_bf, w1a_bf[0:3, :],
                 preferred_element_type=jnp.float32) + b1a[...]
    bj = jnp.dot(pj_bf, w1a_bf[3:6, :],
                 preferred_element_type=jnp.float32)

    bjn = pj.shape[0]
    n2i = jnp.sum(pi * pi, axis=1, keepdims=True)
    g = jax.lax.dot_general(pi_bf, pj_bf, (((1,), (1,)), ((), ())),
                            preferred_element_type=jnp.float32)
    d2 = n2i + n2jr[0] - 2.0 * g
    mf = ((d2 <= _R2) & (hx_is[...] != hx_jc[0])).astype(jnp.float32)
    pen = (mf - 1.0) * 1e30

    shape3 = (bi, bjn, 128)
    x = (jax.lax.broadcast_in_dim(ai, shape3, (0, 2))
         + jax.lax.broadcast_in_dim(bj, shape3, (1, 2))
         + jax.lax.broadcast_in_dim(pen, shape3, (0, 1)))
    hm = jnp.maximum(x, 0.0)
    acc = jnp.sum(hm, axis=0)
    acc2 = jnp.sum(hm * hm, axis=0)
    ccnt = jnp.sum(mf, axis=0)

    @pl.when(ib == 0)
    def _():
        hcol[...] = jnp.zeros_like(hcol)
        cnt[...] = jnp.zeros_like(cnt)

    @pl.when((ib == 0) & (jb == 0))
    def _():
        sumh[...] = jnp.zeros_like(sumh)
        sumh2[...] = jnp.zeros_like(sumh2)

    hcol[...] += acc
    cnt[...] += ccnt.reshape(1, 1, -1)
    sumh[...] += jnp.sum(acc, axis=0, keepdims=True)
    sumh2[...] += jnp.sum(acc2, axis=0, keepdims=True)


def _phase2_body(pos, hcol, cnt, sumh, sumh2, w1b, b1b, g1, be1,
                 w2a, b2a, g2, be2, w2b, b2b, out):
    e = jnp.sum(cnt[...])
    m = sumh[...] / e
    v = sumh2[...] / e - m * m
    scale = g1[...] * jax.lax.rsqrt(v + 1e-5)
    ceff = jnp.dot(be1[...] - m * scale, w1b[...],
                   preferred_element_type=jnp.float32) + b1b[...]
    s = jnp.dot(hcol[...] * scale, w1b[...],
                preferred_element_type=jnp.float32) + cnt[...] * ceff
    u = s / jnp.maximum(cnt[...], 1.0)

    w2a_bf = _bf(w2a[...])
    t = (jnp.dot(_bf(pos[...]), w2a_bf[0:3, :],
                 preferred_element_type=jnp.float32)
         + jnp.dot(_bf(u), w2a_bf[3:6, :],
                   preferred_element_type=jnp.float32)
         + b2a[...])
    t = jnp.maximum(t, 0.0)
    m2 = jnp.mean(t, axis=0, keepdims=True)
    d = t - m2
    v2 = jnp.mean(d * d, axis=0, keepdims=True)
    tn = d * jax.lax.rsqrt(v2 + 1e-5) * g2[...] + be2[...]
    out[...] = jnp.dot(_bf(tn), _bf(w2b[...]),
                       preferred_element_type=jnp.float32) + b2b[...]


def kernel(pos, helix, W1a, b1a, g1, be1, W1b, b1b, W2a, b2a, g2, be2, W2b, b2b):
    n = pos.shape[0]
    nj = n // _BJ
    ni = n // _BI
    hx_c = helix.reshape(n, 1)
    hx_r = helix.reshape(nj, 1, _BJ)
    n2r = jnp.sum(pos * pos, axis=1).reshape(nj, 1, _BJ)
    b1a2 = b1a.reshape(1, -1)

    hcol, cnt, sumh, sumh2 = pl.pallas_call(
        _phase1_body,
        grid=(nj, ni),
        in_specs=[
            pl.BlockSpec((_BI, 3), lambda jb, ib: (ib, 0)),
            pl.BlockSpec((_BJ, 3), lambda jb, ib: (jb, 0)),
            pl.BlockSpec((_BI, 1), lambda jb, ib: (ib, 0)),
            pl.BlockSpec((1, 1, _BJ), lambda jb, ib: (jb, 0, 0)),
            pl.BlockSpec((1, 1, _BJ), lambda jb, ib: (jb, 0, 0)),
            pl.BlockSpec((6, 128), lambda jb, ib: (0, 0)),
            pl.BlockSpec((1, 128), lambda jb, ib: (0, 0)),
        ],
        out_specs=[
            pl.BlockSpec((_BJ, 128), lambda jb, ib: (jb, 0)),
            pl.BlockSpec((1, 1, _BJ), lambda jb, ib: (jb, 0, 0)),
            pl.BlockSpec((1, 128), lambda jb, ib: (0, 0)),
            pl.BlockSpec((1, 128), lambda jb, ib: (0, 0)),
        ],
        out_shape=[
            jax.ShapeDtypeStruct((n, 128), jnp.float32),
            jax.ShapeDtypeStruct((nj, 1, _BJ), jnp.float32),
            jax.ShapeDtypeStruct((1, 128), jnp.float32),
            jax.ShapeDtypeStruct((1, 128), jnp.float32),
        ],
        compiler_params=pltpu.CompilerParams(
            dimension_semantics=("arbitrary", "arbitrary")),
    )(pos, pos, hx_c, hx_r, n2r, W1a, b1a2)

    out = pl.pallas_call(
        _phase2_body,
        out_shape=jax.ShapeDtypeStruct((n, 3), jnp.float32),
    )(pos, hcol, cnt.reshape(n, 1), sumh, sumh2,
      W1b, b1b.reshape(1, -1), g1.reshape(1, -1), be1.reshape(1, -1),
      W2a, b2a.reshape(1, -1), g2.reshape(1, -1), be2.reshape(1, -1),
      W2b, b2b.reshape(1, -1))
    return out


# speedup vs baseline: 27.4309x; 2.9938x over previous
import jax
import jax.numpy as jnp
from jax.experimental import pallas as pl
from jax.experimental.pallas import tpu as pltpu

_R2 = 6.25


def _bf(x):
    return x.astype(jnp.bfloat16).astype(jnp.float32)
_BI = 200
_BJ = 200


def _phase1_body(stats, pos_i, pos_j, hx_is, hx_jc, n2jr, w1a, b1a,
                 hcol, cnt, sumh, sumh2):
    jb = pl.program_id(0)
    ib = pl.program_id(1)

    @pl.when(ib == 0)
    def _():
        hcol[...] = jnp.zeros_like(hcol)
        cnt[...] = jnp.zeros_like(cnt)

    @pl.when((ib == 0) & (jb == 0))
    def _():
        sumh[...] = jnp.zeros_like(sumh)
        sumh2[...] = jnp.zeros_like(sumh2)

    gap = jnp.maximum(jnp.maximum(stats[0, jb] - stats[1, ib],
                                  stats[0, ib] - stats[1, jb]), 0.0)
    slack = (stats[2, ib] + stats[2, jb]) * 0.00390625 + 0.5

    @pl.when(gap * gap <= _R2 + slack)
    def _compute():
        _phase1_tile(pos_i, pos_j, hx_is, hx_jc, n2jr, w1a, b1a,
                     hcol, cnt, sumh, sumh2)


def _phase1_tile(pos_i, pos_j, hx_is, hx_jc, n2jr, w1a, b1a,
                 hcol, cnt, sumh, sumh2):
    pi = pos_i[...]
    pj = pos_j[...]
    bi = pi.shape[0]
    w1a_bf = _bf(w1a[...])
    pi_bf = _bf(pi)
    pj_bf = _bf(pj)
    ai = jnp.dot(pi_bf, w1a_bf[0:3, :],
                 preferred_element_type=jnp.float32) + b1a[...]
    bj = jnp.dot(pj_bf, w1a_bf[3:6, :],
                 preferred_element_type=jnp.float32)

    bjn = pj.shape[0]
    n2i = jnp.sum(pi * pi, axis=1, keepdims=True)
    g = jax.lax.dot_general(pi_bf, pj_bf, (((1,), (1,)), ((), ())),
                            preferred_element_type=jnp.float32)
    d2 = n2i + n2jr[0] - 2.0 * g
    mf = ((d2 <= _R2) & (hx_is[...] != hx_jc[0])).astype(jnp.float32)
    pen = (mf - 1.0) * 1e30

    shape3 = (bi, bjn, 128)
    x = (jax.lax.broadcast_in_dim(ai, shape3, (0, 2))
         + jax.lax.broadcast_in_dim(bj, shape3, (1, 2))
         + jax.lax.broadcast_in_dim(pen, shape3, (0, 1)))
    hm = jnp.maximum(x, 0.0)
    acc = jnp.sum(hm, axis=0)
    acc2 = jnp.sum(hm * hm, axis=0)
    ccnt = jnp.sum(mf, axis=0)

    hcol[...] += acc
    cnt[...] += ccnt.reshape(1, 1, -1)
    sumh[...] += jnp.sum(acc, axis=0, keepdims=True)
    sumh2[...] += jnp.sum(acc2, axis=0, keepdims=True)


def _phase2_body(pos, hcol, cnt, sumh, sumh2, w1b, b1b, g1, be1,
                 w2a, b2a, g2, be2, w2b, b2b, out):
    e = jnp.sum(cnt[...])
    m = sumh[...] / e
    v = sumh2[...] / e - m * m
    scale = g1[...] * jax.lax.rsqrt(v + 1e-5)
    ceff = jnp.dot(be1[...] - m * scale, w1b[...],
                   preferred_element_type=jnp.float32) + b1b[...]
    s = jnp.dot(hcol[...] * scale, w1b[...],
                preferred_element_type=jnp.float32) + cnt[...] * ceff
    u = s / jnp.maximum(cnt[...], 1.0)

    w2a_bf = _bf(w2a[...])
    t = (jnp.dot(_bf(pos[...]), w2a_bf[0:3, :],
                 preferred_element_type=jnp.float32)
         + jnp.dot(_bf(u), w2a_bf[3:6, :],
                   preferred_element_type=jnp.float32)
         + b2a[...])
    t = jnp.maximum(t, 0.0)
    m2 = jnp.mean(t, axis=0, keepdims=True)
    d = t - m2
    v2 = jnp.mean(d * d, axis=0, keepdims=True)
    tn = d * jax.lax.rsqrt(v2 + 1e-5) * g2[...] + be2[...]
    out[...] = jnp.dot(_bf(tn), _bf(w2b[...]),
                       preferred_element_type=jnp.float32) + b2b[...]


def kernel(pos, helix, W1a, b1a, g1, be1, W1b, b1b, W2a, b2a, g2, be2, W2b, b2b):
    n = pos.shape[0]
    nj = n // _BJ
    ni = n // _BI
    perm = jnp.argsort(pos[:, 0])
    pos = pos[perm]
    helix = helix[perm]
    hx_c = helix.reshape(n, 1)
    hx_r = helix.reshape(nj, 1, _BJ)
    n2v = jnp.sum(pos * pos, axis=1)
    n2r = n2v.reshape(nj, 1, _BJ)
    xs = pos[:, 0].reshape(ni, _BI)
    stats = jnp.stack([jnp.min(xs, axis=1), jnp.max(xs, axis=1),
                       jnp.max(n2v.reshape(ni, _BI), axis=1)], axis=0)
    b1a2 = b1a.reshape(1, -1)

    hcol, cnt, sumh, sumh2 = pl.pallas_call(
        _phase1_body,
        grid=(nj, ni),
        in_specs=[
            pl.BlockSpec((3, ni), lambda jb, ib: (0, 0),
                         memory_space=pltpu.SMEM),
            pl.BlockSpec((_BI, 3), lambda jb, ib: (ib, 0)),
            pl.BlockSpec((_BJ, 3), lambda jb, ib: (jb, 0)),
            pl.BlockSpec((_BI, 1), lambda jb, ib: (ib, 0)),
            pl.BlockSpec((1, 1, _BJ), lambda jb, ib: (jb, 0, 0)),
            pl.BlockSpec((1, 1, _BJ), lambda jb, ib: (jb, 0, 0)),
            pl.BlockSpec((6, 128), lambda jb, ib: (0, 0)),
            pl.BlockSpec((1, 128), lambda jb, ib: (0, 0)),
        ],
        out_specs=[
            pl.BlockSpec((_BJ, 128), lambda jb, ib: (jb, 0)),
            pl.BlockSpec((1, 1, _BJ), lambda jb, ib: (jb, 0, 0)),
            pl.BlockSpec((1, 128), lambda jb, ib: (0, 0)),
            pl.BlockSpec((1, 128), lambda jb, ib: (0, 0)),
        ],
        out_shape=[
            jax.ShapeDtypeStruct((n, 128), jnp.float32),
            jax.ShapeDtypeStruct((nj, 1, _BJ), jnp.float32),
            jax.ShapeDtypeStruct((1, 128), jnp.float32),
            jax.ShapeDtypeStruct((1, 128), jnp.float32),
        ],
        compiler_params=pltpu.CompilerParams(
            dimension_semantics=("arbitrary", "arbitrary")),
    )(stats, pos, pos, hx_c, hx_r, n2r, W1a, b1a2)

    out = pl.pallas_call(
        _phase2_body,
        out_shape=jax.ShapeDtypeStruct((n, 3), jnp.float32),
    )(pos, hcol, cnt.reshape(n, 1), sumh, sumh2,
      W1b, b1b.reshape(1, -1), g1.reshape(1, -1), be1.reshape(1, -1),
      W2a, b2a.reshape(1, -1), g2.reshape(1, -1), be2.reshape(1, -1),
      W2b, b2b.reshape(1, -1))
    inv = jnp.zeros((n,), jnp.int32).at[perm].set(
        jnp.arange(n, dtype=jnp.int32))
    return out[inv]


# speedup vs baseline: 30.4107x; 1.1086x over previous
import jax
import jax.numpy as jnp
from jax.experimental import pallas as pl
from jax.experimental.pallas import tpu as pltpu

_R2 = 6.25


def _bf(x):
    return x.astype(jnp.bfloat16).astype(jnp.float32)
_BIO = 1000
_CH = 40
_NCH = _BIO // _CH
_BJ = 200


def _phase1_body(stats_f, stats_j, pos_i, pos_j, hx_is, hx_jc, n2jr,
                 w1a, b1a, hcol, cnt, sumh, sumh2):
    jb = pl.program_id(0)
    ib = pl.program_id(1)

    @pl.when(ib == 0)
    def _():
        hcol[...] = jnp.zeros_like(hcol)
        cnt[...] = jnp.zeros_like(cnt)

    @pl.when((ib == 0) & (jb == 0))
    def _():
        sumh[...] = jnp.zeros_like(sumh)
        sumh2[...] = jnp.zeros_like(sumh2)

    pj = pos_j[...]
    w1a_bf = _bf(w1a[...])
    pj_bf = _bf(pj)
    bj = jnp.dot(pj_bf, w1a_bf[3:6, :],
                 preferred_element_type=jnp.float32)
    n2j = n2jr[0]
    hxj = hx_jc[0]
    xj_min = stats_j[0, jb]
    xj_max = stats_j[1, jb]
    n2j_max = stats_j[2, jb]
    shape3 = (_CH, pj.shape[0], 128)

    def chunk(c, carry):
        gc = ib * _NCH + c
        gap = jnp.maximum(jnp.maximum(xj_min - stats_f[1, gc],
                                      stats_f[0, gc] - xj_max), 0.0)
        slack = (stats_f[2, gc] + n2j_max) * 0.00390625 + 0.5

        @pl.when(gap * gap <= _R2 + slack)
        def _():
            pi = pos_i[pl.ds(c * _CH, _CH), :]
            hxi = hx_is[pl.ds(c * _CH, _CH), :]
            pi_bf = _bf(pi)
            ai = jnp.dot(pi_bf, w1a_bf[0:3, :],
                         preferred_element_type=jnp.float32) + b1a[...]
            n2i = jnp.sum(pi * pi, axis=1, keepdims=True)
            g = jax.lax.dot_general(pi_bf, pj_bf, (((1,), (1,)), ((), ())),
                                    preferred_element_type=jnp.float32)
            d2 = n2i + n2j - 2.0 * g
            mf = ((d2 <= _R2) & (hxi != hxj)).astype(jnp.float32)
            pen = (mf - 1.0) * 1e30
            x = (jax.lax.broadcast_in_dim(ai, shape3, (0, 2))
                 + jax.lax.broadcast_in_dim(bj, shape3, (1, 2))
                 + jax.lax.broadcast_in_dim(pen, shape3, (0, 1)))
            hm = jnp.maximum(x, 0.0)
            cs = jnp.sum(hm, axis=0)
            a2 = jnp.sum(hm * hm, axis=0)
            hcol[...] += cs
            cnt[...] += jnp.sum(mf, axis=0).reshape(1, 1, -1)
            sumh[...] += jnp.sum(cs, axis=0, keepdims=True)
            sumh2[...] += jnp.sum(a2, axis=0, keepdims=True)

        return carry

    jax.lax.fori_loop(0, _NCH, chunk, 0)


def _phase2_body(pos, hcol, cnt, sumh, sumh2, w1b, b1b, g1, be1,
                 w2a, b2a, g2, be2, w2b, b2b, out):
    e = jnp.sum(cnt[...])
    m = sumh[...] / e
    v = sumh2[...] / e - m * m
    scale = g1[...] * jax.lax.rsqrt(v + 1e-5)
    ceff = jnp.dot(be1[...] - m * scale, w1b[...],
                   preferred_element_type=jnp.float32) + b1b[...]
    s = jnp.dot(hcol[...] * scale, w1b[...],
                preferred_element_type=jnp.float32) + cnt[...] * ceff
    u = s / jnp.maximum(cnt[...], 1.0)

    w2a_bf = _bf(w2a[...])
    t = (jnp.dot(_bf(pos[...]), w2a_bf[0:3, :],
                 preferred_element_type=jnp.float32)
         + jnp.dot(_bf(u), w2a_bf[3:6, :],
                   preferred_element_type=jnp.float32)
         + b2a[...])
    t = jnp.maximum(t, 0.0)
    m2 = jnp.mean(t, axis=0, keepdims=True)
    d = t - m2
    v2 = jnp.mean(d * d, axis=0, keepdims=True)
    tn = d * jax.lax.rsqrt(v2 + 1e-5) * g2[...] + be2[...]
    out[...] = jnp.dot(_bf(tn), _bf(w2b[...]),
                       preferred_element_type=jnp.float32) + b2b[...]


def kernel(pos, helix, W1a, b1a, g1, be1, W1b, b1b, W2a, b2a, g2, be2, W2b, b2b):
    n = pos.shape[0]
    nj = n // _BJ
    ni = n // _BIO
    nch = n // _CH
    perm = jnp.argsort(pos[:, 0])
    pos = pos[perm]
    helix = helix[perm]
    hx_c = helix.reshape(n, 1)
    hx_r = helix.reshape(nj, 1, _BJ)
    n2v = jnp.sum(pos * pos, axis=1)
    n2r = n2v.reshape(nj, 1, _BJ)
    xf = pos[:, 0].reshape(nch, _CH)
    stats_f = jnp.stack([jnp.min(xf, axis=1), jnp.max(xf, axis=1),
                         jnp.max(n2v.reshape(nch, _CH), axis=1)], axis=0)
    xj = pos[:, 0].reshape(nj, _BJ)
    stats_j = jnp.stack([jnp.min(xj, axis=1), jnp.max(xj, axis=1),
                         jnp.max(n2v.reshape(nj, _BJ), axis=1)], axis=0)
    b1a2 = b1a.reshape(1, -1)

    hcol, cnt, sumh, sumh2 = pl.pallas_call(
        _phase1_body,
        grid=(nj, ni),
        in_specs=[
            pl.BlockSpec((3, nch), lambda jb, ib: (0, 0),
                         memory_space=pltpu.SMEM),
            pl.BlockSpec((3, nj), lambda jb, ib: (0, 0),
                         memory_space=pltpu.SMEM),
            pl.BlockSpec((_BIO, 3), lambda jb, ib: (ib, 0)),
            pl.BlockSpec((_BJ, 3), lambda jb, ib: (jb, 0)),
            pl.BlockSpec((_BIO, 1), lambda jb, ib: (ib, 0)),
            pl.BlockSpec((1, 1, _BJ), lambda jb, ib: (jb, 0, 0)),
            pl.BlockSpec((1, 1, _BJ), lambda jb, ib: (jb, 0, 0)),
            pl.BlockSpec((6, 128), lambda jb, ib: (0, 0)),
            pl.BlockSpec((1, 128), lambda jb, ib: (0, 0)),
        ],
        out_specs=[
            pl.BlockSpec((_BJ, 128), lambda jb, ib: (jb, 0)),
            pl.BlockSpec((1, 1, _BJ), lambda jb, ib: (jb, 0, 0)),
            pl.BlockSpec((1, 128), lambda jb, ib: (0, 0)),
            pl.BlockSpec((1, 128), lambda jb, ib: (0, 0)),
        ],
        out_shape=[
            jax.ShapeDtypeStruct((n, 128), jnp.float32),
            jax.ShapeDtypeStruct((nj, 1, _BJ), jnp.float32),
            jax.ShapeDtypeStruct((1, 128), jnp.float32),
            jax.ShapeDtypeStruct((1, 128), jnp.float32),
        ],
        compiler_params=pltpu.CompilerParams(
            dimension_semantics=("arbitrary", "arbitrary")),
    )(stats_f, stats_j, pos, pos, hx_c, hx_r, n2r, W1a, b1a2)

    out = pl.pallas_call(
        _phase2_body,
        out_shape=jax.ShapeDtypeStruct((n, 3), jnp.float32),
    )(pos, hcol, cnt.reshape(n, 1), sumh, sumh2,
      W1b, b1b.reshape(1, -1), g1.reshape(1, -1), be1.reshape(1, -1),
      W2a, b2a.reshape(1, -1), g2.reshape(1, -1), be2.reshape(1, -1),
      W2b, b2b.reshape(1, -1))
    inv = jnp.zeros((n,), jnp.int32).at[perm].set(
        jnp.arange(n, dtype=jnp.int32))
    return out[inv]


# speedup vs baseline: 33.6505x; 1.1065x over previous
import jax
import jax.numpy as jnp
from jax.experimental import pallas as pl
from jax.experimental.pallas import tpu as pltpu

_R2 = 6.25


def _bf(x):
    return x.astype(jnp.bfloat16).astype(jnp.float32)
_BIO = 2000
_CH = 40
_NCH = _BIO // _CH
_BJ = 80
_W = 4.0


def _phase1_body(stats_f, stats_j, pos_i, pos_j, hx_is, hx_jc, n2jr,
                 w1a, b1a, hcol, cnt, sumh, sumh2):
    jb = pl.program_id(0)
    ib = pl.program_id(1)

    @pl.when(ib == 0)
    def _():
        hcol[...] = jnp.zeros_like(hcol)
        cnt[...] = jnp.zeros_like(cnt)

    @pl.when((ib == 0) & (jb == 0))
    def _():
        sumh[...] = jnp.zeros_like(sumh)
        sumh2[...] = jnp.zeros_like(sumh2)

    pj = pos_j[...]
    w1a_bf = _bf(w1a[...])
    pj_bf = _bf(pj)
    bj = jnp.dot(pj_bf, w1a_bf[3:6, :],
                 preferred_element_type=jnp.float32)
    n2j = n2jr[0]
    hxj = hx_jc[0]
    xj_min = stats_j[0, jb]
    xj_max = stats_j[1, jb]
    yj_min = stats_j[2, jb]
    yj_max = stats_j[3, jb]
    n2j_max = stats_j[4, jb]
    shape3 = (_CH, pj.shape[0], 128)

    def chunk(c, carry):
        gc = ib * _NCH + c
        xgap = jnp.maximum(jnp.maximum(xj_min - stats_f[1, gc],
                                       stats_f[0, gc] - xj_max), 0.0)
        ygap = jnp.maximum(jnp.maximum(yj_min - stats_f[3, gc],
                                       stats_f[2, gc] - yj_max), 0.0)
        slack = (stats_f[4, gc] + n2j_max) * 0.00390625 + 0.5

        @pl.when(xgap * xgap + ygap * ygap <= _R2 + slack)
        def _():
            pi = pos_i[pl.ds(c * _CH, _CH), :]
            hxi = hx_is[pl.ds(c * _CH, _CH), :]
            pi_bf = _bf(pi)
            ai = jnp.dot(pi_bf, w1a_bf[0:3, :],
                         preferred_element_type=jnp.float32) + b1a[...]
            n2i = jnp.sum(pi * pi, axis=1, keepdims=True)
            g = jax.lax.dot_general(pi_bf, pj_bf, (((1,), (1,)), ((), ())),
                                    preferred_element_type=jnp.float32)
            d2 = n2i + n2j - 2.0 * g
            mf = ((d2 <= _R2) & (hxi != hxj)).astype(jnp.float32)
            pen = (mf - 1.0) * 1e30
            x = (jax.lax.broadcast_in_dim(ai, shape3, (0, 2))
                 + jax.lax.broadcast_in_dim(bj, shape3, (1, 2))
                 + jax.lax.broadcast_in_dim(pen, shape3, (0, 1)))
            hm = jnp.maximum(x, 0.0)
            cs = jnp.sum(hm, axis=0)
            a2 = jnp.sum(hm * hm, axis=0)
            hcol[...] += cs
            cnt[...] += jnp.sum(mf, axis=0).reshape(1, 1, -1)
            sumh[...] += jnp.sum(cs, axis=0, keepdims=True)
            sumh2[...] += jnp.sum(a2, axis=0, keepdims=True)

        return carry

    jax.lax.fori_loop(0, _NCH, chunk, 0)


def _phase2_body(pos, hcol, cnt, sumh, sumh2, w1b, b1b, g1, be1,
                 w2a, b2a, g2, be2, w2b, b2b, out):
    e = jnp.sum(cnt[...])
    m = sumh[...] / e
    v = sumh2[...] / e - m * m
    scale = g1[...] * jax.lax.rsqrt(v + 1e-5)
    ceff = jnp.dot(be1[...] - m * scale, w1b[...],
                   preferred_element_type=jnp.float32) + b1b[...]
    s = jnp.dot(hcol[...] * scale, w1b[...],
                preferred_element_type=jnp.float32) + cnt[...] * ceff
    u = s / jnp.maximum(cnt[...], 1.0)

    w2a_bf = _bf(w2a[...])
    t = (jnp.dot(_bf(pos[...]), w2a_bf[0:3, :],
                 preferred_element_type=jnp.float32)
         + jnp.dot(_bf(u), w2a_bf[3:6, :],
                   preferred_element_type=jnp.float32)
         + b2a[...])
    t = jnp.maximum(t, 0.0)
    m2 = jnp.mean(t, axis=0, keepdims=True)
    d = t - m2
    v2 = jnp.mean(d * d, axis=0, keepdims=True)
    tn = d * jax.lax.rsqrt(v2 + 1e-5) * g2[...] + be2[...]
    out[...] = jnp.dot(_bf(tn), _bf(w2b[...]),
                       preferred_element_type=jnp.float32) + b2b[...]


def kernel(pos, helix, W1a, b1a, g1, be1, W1b, b1b, W2a, b2a, g2, be2, W2b, b2b):
    n = pos.shape[0]
    nj = n // _BJ
    ni = n // _BIO
    nch = n // _CH
    perm = jnp.argsort(jnp.floor(pos[:, 0] / _W) * 256.0 + pos[:, 1])
    pos = pos[perm]
    helix = helix[perm]
    hx_c = helix.reshape(n, 1)
    hx_r = helix.reshape(nj, 1, _BJ)
    n2v = jnp.sum(pos * pos, axis=1)
    n2r = n2v.reshape(nj, 1, _BJ)
    xf = pos[:, 0].reshape(nch, _CH)
    yf = pos[:, 1].reshape(nch, _CH)
    stats_f = jnp.stack([jnp.min(xf, axis=1), jnp.max(xf, axis=1),
                         jnp.min(yf, axis=1), jnp.max(yf, axis=1),
                         jnp.max(n2v.reshape(nch, _CH), axis=1)], axis=0)
    xj = pos[:, 0].reshape(nj, _BJ)
    yj = pos[:, 1].reshape(nj, _BJ)
    stats_j = jnp.stack([jnp.min(xj, axis=1), jnp.max(xj, axis=1),
                         jnp.min(yj, axis=1), jnp.max(yj, axis=1),
                         jnp.max(n2v.reshape(nj, _BJ), axis=1)], axis=0)
    b1a2 = b1a.reshape(1, -1)

    hcol, cnt, sumh, sumh2 = pl.pallas_call(
        _phase1_body,
        grid=(nj, ni),
        in_specs=[
            pl.BlockSpec((5, nch), lambda jb, ib: (0, 0),
                         memory_space=pltpu.SMEM),
            pl.BlockSpec((5, nj), lambda jb, ib: (0, 0),
                         memory_space=pltpu.SMEM),
            pl.BlockSpec((_BIO, 3), lambda jb, ib: (ib, 0)),
            pl.BlockSpec((_BJ, 3), lambda jb, ib: (jb, 0)),
            pl.BlockSpec((_BIO, 1), lambda jb, ib: (ib, 0)),
            pl.BlockSpec((1, 1, _BJ), lambda jb, ib: (jb, 0, 0)),
            pl.BlockSpec((1, 1, _BJ), lambda jb, ib: (jb, 0, 0)),
            pl.BlockSpec((6, 128), lambda jb, ib: (0, 0)),
            pl.BlockSpec((1, 128), lambda jb, ib: (0, 0)),
        ],
        out_specs=[
            pl.BlockSpec((_BJ, 128), lambda jb, ib: (jb, 0)),
            pl.BlockSpec((1, 1, _BJ), lambda jb, ib: (jb, 0, 0)),
            pl.BlockSpec((1, 128), lambda jb, ib: (0, 0)),
            pl.BlockSpec((1, 128), lambda jb, ib: (0, 0)),
        ],
        out_shape=[
            jax.ShapeDtypeStruct((n, 128), jnp.float32),
            jax.ShapeDtypeStruct((nj, 1, _BJ), jnp.float32),
            jax.ShapeDtypeStruct((1, 128), jnp.float32),
            jax.ShapeDtypeStruct((1, 128), jnp.float32),
        ],
        compiler_params=pltpu.CompilerParams(
            dimension_semantics=("arbitrary", "arbitrary")),
    )(stats_f, stats_j, pos, pos, hx_c, hx_r, n2r, W1a, b1a2)

    out = pl.pallas_call(
        _phase2_body,
        out_shape=jax.ShapeDtypeStruct((n, 3), jnp.float32),
    )(pos, hcol, cnt.reshape(n, 1), sumh, sumh2,
      W1b, b1b.reshape(1, -1), g1.reshape(1, -1), be1.reshape(1, -1),
      W2a, b2a.reshape(1, -1), g2.reshape(1, -1), be2.reshape(1, -1),
      W2b, b2b.reshape(1, -1))
    inv = jnp.zeros((n,), jnp.int32).at[perm].set(
        jnp.arange(n, dtype=jnp.int32))
    return out[inv]


# speedup vs baseline: 35.6095x; 1.0582x over previous
import jax
import jax.numpy as jnp
from jax.experimental import pallas as pl
from jax.experimental.pallas import tpu as pltpu

_R2 = 6.25


def _bf(x):
    return x.astype(jnp.bfloat16).astype(jnp.float32)
_BIO = 10000
_CH = 40
_NCH = _BIO // _CH
_BJ = 80
_W = 4.0


def _phase1_body(stats_f, stats_j, pos_i, pos_j, hx_is, hx_jc, n2jr,
                 w1a, b1a, hcol, cnt, sumh, sumh2):
    jb = pl.program_id(0)
    ib = pl.program_id(1)

    @pl.when(ib == 0)
    def _():
        hcol[...] = jnp.zeros_like(hcol)
        cnt[...] = jnp.zeros_like(cnt)

    @pl.when((ib == 0) & (jb == 0))
    def _():
        sumh[...] = jnp.zeros_like(sumh)
        sumh2[...] = jnp.zeros_like(sumh2)

    pj = pos_j[...]
    w1a_bf = _bf(w1a[...])
    pj_bf = _bf(pj)
    bj = jnp.dot(pj_bf, w1a_bf[3:6, :],
                 preferred_element_type=jnp.float32)
    n2j = n2jr[0]
    hxj = hx_jc[0]
    xj_min = stats_j[0, jb]
    xj_max = stats_j[1, jb]
    yj_min = stats_j[2, jb]
    yj_max = stats_j[3, jb]
    n2j_max = stats_j[4, jb]
    shape3 = (_CH, pj.shape[0], 128)

    def chunk(c, carry):
        gc = ib * _NCH + c
        xgap = jnp.maximum(jnp.maximum(xj_min - stats_f[1, gc],
                                       stats_f[0, gc] - xj_max), 0.0)
        ygap = jnp.maximum(jnp.maximum(yj_min - stats_f[3, gc],
                                       stats_f[2, gc] - yj_max), 0.0)
        slack = (stats_f[4, gc] + n2j_max) * 0.00390625 + 0.5

        @pl.when(xgap * xgap + ygap * ygap <= _R2 + slack)
        def _():
            pi = pos_i[pl.ds(c * _CH, _CH), :]
            hxi = hx_is[pl.ds(c * _CH, _CH), :]
            pi_bf = _bf(pi)
            ai = jnp.dot(pi_bf, w1a_bf[0:3, :],
                         preferred_element_type=jnp.float32) + b1a[...]
            n2i = jnp.sum(pi * pi, axis=1, keepdims=True)
            g = jax.lax.dot_general(pi_bf, pj_bf, (((1,), (1,)), ((), ())),
                                    preferred_element_type=jnp.float32)
            d2 = n2i + n2j - 2.0 * g
            mf = ((d2 <= _R2) & (hxi != hxj)).astype(jnp.float32)
            pen = (mf - 1.0) * 1e30
            x = (jax.lax.broadcast_in_dim(ai, shape3, (0, 2))
                 + jax.lax.broadcast_in_dim(bj, shape3, (1, 2))
                 + jax.lax.broadcast_in_dim(pen, shape3, (0, 1)))
            hm = jnp.maximum(x, 0.0)
            cs = jnp.sum(hm, axis=0)
            a2 = jnp.sum(hm * hm, axis=0)
            hcol[...] += cs
            cnt[...] += jnp.sum(mf, axis=0).reshape(1, 1, -1)
            sumh[...] += jnp.sum(cs, axis=0, keepdims=True)
            sumh2[...] += jnp.sum(a2, axis=0, keepdims=True)

        return carry

    jax.lax.fori_loop(0, _NCH, chunk, 0)


def _phase2_body(pos, hcol, cnt, sumh, sumh2, w1b, b1b, g1, be1,
                 w2a, b2a, g2, be2, w2b, b2b, out):
    e = jnp.sum(cnt[...])
    m = sumh[...] / e
    v = sumh2[...] / e - m * m
    scale = g1[...] * jax.lax.rsqrt(v + 1e-5)
    ceff = jnp.dot(be1[...] - m * scale, w1b[...],
                   preferred_element_type=jnp.float32) + b1b[...]
    s = jnp.dot(hcol[...] * scale, w1b[...],
                preferred_element_type=jnp.float32) + cnt[...] * ceff
    u = s / jnp.maximum(cnt[...], 1.0)

    w2a_bf = _bf(w2a[...])
    t = (jnp.dot(_bf(pos[...]), w2a_bf[0:3, :],
                 preferred_element_type=jnp.float32)
         + jnp.dot(_bf(u), w2a_bf[3:6, :],
                   preferred_element_type=jnp.float32)
         + b2a[...])
    t = jnp.maximum(t, 0.0)
    m2 = jnp.mean(t, axis=0, keepdims=True)
    d = t - m2
    v2 = jnp.mean(d * d, axis=0, keepdims=True)
    tn = d * jax.lax.rsqrt(v2 + 1e-5) * g2[...] + be2[...]
    out[...] = jnp.dot(_bf(tn), _bf(w2b[...]),
                       preferred_element_type=jnp.float32) + b2b[...]


def kernel(pos, helix, W1a, b1a, g1, be1, W1b, b1b, W2a, b2a, g2, be2, W2b, b2b):
    n = pos.shape[0]
    nj = n // _BJ
    ni = n // _BIO
    nch = n // _CH
    perm = jnp.argsort(jnp.floor(pos[:, 0] / _W) * 256.0 + pos[:, 1])
    pos = pos[perm]
    helix = helix[perm]
    hx_c = helix.reshape(n, 1)
    hx_r = helix.reshape(nj, 1, _BJ)
    n2v = jnp.sum(pos * pos, axis=1)
    n2r = n2v.reshape(nj, 1, _BJ)
    xf = pos[:, 0].reshape(nch, _CH)
    yf = pos[:, 1].reshape(nch, _CH)
    stats_f = jnp.stack([jnp.min(xf, axis=1), jnp.max(xf, axis=1),
                         jnp.min(yf, axis=1), jnp.max(yf, axis=1),
                         jnp.max(n2v.reshape(nch, _CH), axis=1)], axis=0)
    xj = pos[:, 0].reshape(nj, _BJ)
    yj = pos[:, 1].reshape(nj, _BJ)
    stats_j = jnp.stack([jnp.min(xj, axis=1), jnp.max(xj, axis=1),
                         jnp.min(yj, axis=1), jnp.max(yj, axis=1),
                         jnp.max(n2v.reshape(nj, _BJ), axis=1)], axis=0)
    b1a2 = b1a.reshape(1, -1)

    hcol, cnt, sumh, sumh2 = pl.pallas_call(
        _phase1_body,
        grid=(nj, ni),
        in_specs=[
            pl.BlockSpec((5, nch), lambda jb, ib: (0, 0),
                         memory_space=pltpu.SMEM),
            pl.BlockSpec((5, nj), lambda jb, ib: (0, 0),
                         memory_space=pltpu.SMEM),
            pl.BlockSpec((_BIO, 3), lambda jb, ib: (ib, 0)),
            pl.BlockSpec((_BJ, 3), lambda jb, ib: (jb, 0)),
            pl.BlockSpec((_BIO, 1), lambda jb, ib: (ib, 0)),
            pl.BlockSpec((1, 1, _BJ), lambda jb, ib: (jb, 0, 0)),
            pl.BlockSpec((1, 1, _BJ), lambda jb, ib: (jb, 0, 0)),
            pl.BlockSpec((6, 128), lambda jb, ib: (0, 0)),
            pl.BlockSpec((1, 128), lambda jb, ib: (0, 0)),
        ],
        out_specs=[
            pl.BlockSpec((_BJ, 128), lambda jb, ib: (jb, 0)),
            pl.BlockSpec((1, 1, _BJ), lambda jb, ib: (jb, 0, 0)),
            pl.BlockSpec((1, 128), lambda jb, ib: (0, 0)),
            pl.BlockSpec((1, 128), lambda jb, ib: (0, 0)),
        ],
        out_shape=[
            jax.ShapeDtypeStruct((n, 128), jnp.float32),
            jax.ShapeDtypeStruct((nj, 1, _BJ), jnp.float32),
            jax.ShapeDtypeStruct((1, 128), jnp.float32),
            jax.ShapeDtypeStruct((1, 128), jnp.float32),
        ],
        compiler_params=pltpu.CompilerParams(
            dimension_semantics=("arbitrary", "arbitrary")),
    )(stats_f, stats_j, pos, pos, hx_c, hx_r, n2r, W1a, b1a2)

    out = pl.pallas_call(
        _phase2_body,
        out_shape=jax.ShapeDtypeStruct((n, 3), jnp.float32),
    )(pos, hcol, cnt.reshape(n, 1), sumh, sumh2,
      W1b, b1b.reshape(1, -1), g1.reshape(1, -1), be1.reshape(1, -1),
      W2a, b2a.reshape(1, -1), g2.reshape(1, -1), be2.reshape(1, -1),
      W2b, b2b.reshape(1, -1))
    inv = jnp.zeros((n,), jnp.int32).at[perm].set(
        jnp.arange(n, dtype=jnp.int32))
    return out[inv]


# speedup vs baseline: 38.8430x; 1.0908x over previous
import jax
import jax.numpy as jnp
from jax.experimental import pallas as pl
from jax.experimental.pallas import tpu as pltpu

_R2 = 6.25


def _bf(x):
    return x.astype(jnp.bfloat16).astype(jnp.float32)
_BIO = 10000
_CH = 40
_NCH = _BIO // _CH
_BJ = 80
_W = 4.0


def _phase1_body(stats_f, stats_j, pos_i, pos_j, hx_is, hx_jc, n2jr,
                 w1a, b1a, hcol, cnt, sumh, sumh2):
    jb = pl.program_id(0)
    ib = pl.program_id(1)

    @pl.when(ib == 0)
    def _():
        hcol[...] = jnp.zeros_like(hcol)
        cnt[...] = jnp.zeros_like(cnt)

    @pl.when((ib == 0) & (jb == 0))
    def _():
        sumh[...] = jnp.zeros_like(sumh)
        sumh2[...] = jnp.zeros_like(sumh2)

    pj = pos_j[...]
    w1a_bf = _bf(w1a[...])
    pj_bf = _bf(pj)
    bj = jnp.dot(pj_bf, w1a_bf[3:6, :],
                 preferred_element_type=jnp.float32)
    bj16 = bj.astype(jnp.bfloat16)
    n2j = n2jr[0]
    hxj = hx_jc[0]
    xj_min = stats_j[0, jb]
    xj_max = stats_j[1, jb]
    yj_min = stats_j[2, jb]
    yj_max = stats_j[3, jb]
    n2j_max = stats_j[4, jb]
    shape3 = (_CH, pj.shape[0], 128)

    def chunk(c, carry):
        gc = ib * _NCH + c
        xgap = jnp.maximum(jnp.maximum(xj_min - stats_f[1, gc],
                                       stats_f[0, gc] - xj_max), 0.0)
        ygap = jnp.maximum(jnp.maximum(yj_min - stats_f[3, gc],
                                       stats_f[2, gc] - yj_max), 0.0)
        slack = (stats_f[4, gc] + n2j_max) * 0.00390625 + 0.5

        @pl.when(xgap * xgap + ygap * ygap <= _R2 + slack)
        def _():
            pi = pos_i[pl.ds(c * _CH, _CH), :]
            hxi = hx_is[pl.ds(c * _CH, _CH), :]
            pi_bf = _bf(pi)
            ai = jnp.dot(pi_bf, w1a_bf[0:3, :],
                         preferred_element_type=jnp.float32) + b1a[...]
            n2i = jnp.sum(pi * pi, axis=1, keepdims=True)
            g = jax.lax.dot_general(pi_bf, pj_bf, (((1,), (1,)), ((), ())),
                                    preferred_element_type=jnp.float32)
            d2 = n2i + n2j - 2.0 * g
            mf = ((d2 <= _R2) & (hxi != hxj)).astype(jnp.float32)
            pen = (mf - 1.0) * 1e30
            x = (jax.lax.broadcast_in_dim(ai.astype(jnp.bfloat16), shape3,
                                          (0, 2))
                 + jax.lax.broadcast_in_dim(bj16, shape3, (1, 2))
                 + jax.lax.broadcast_in_dim(pen.astype(jnp.bfloat16), shape3,
                                            (0, 1)))
            hm = jnp.maximum(x, jnp.bfloat16(0.0))
            cs = jnp.sum(hm, axis=0).astype(jnp.float32)
            a2 = jnp.sum(hm * hm, axis=0).astype(jnp.float32)
            hcol[...] += cs
            cnt[...] += jnp.sum(mf, axis=0).reshape(1, 1, -1)
            sumh[...] += jnp.sum(cs, axis=0, keepdims=True)
            sumh2[...] += jnp.sum(a2, axis=0, keepdims=True)

        return carry

    jax.lax.fori_loop(0, _NCH, chunk, 0)


def _phase2_body(pos, hcol, cnt, sumh, sumh2, w1b, b1b, g1, be1,
                 w2a, b2a, g2, be2, w2b, b2b, out):
    e = jnp.sum(cnt[...])
    m = sumh[...] / e
    v = sumh2[...] / e - m * m
    scale = g1[...] * jax.lax.rsqrt(v + 1e-5)
    ceff = jnp.dot(be1[...] - m * scale, w1b[...],
                   preferred_element_type=jnp.float32) + b1b[...]
    s = jnp.dot(hcol[...] * scale, w1b[...],
                preferred_element_type=jnp.float32) + cnt[...] * ceff
    u = s / jnp.maximum(cnt[...], 1.0)

    w2a_bf = _bf(w2a[...])
    t = (jnp.dot(_bf(pos[...]), w2a_bf[0:3, :],
                 preferred_element_type=jnp.float32)
         + jnp.dot(_bf(u), w2a_bf[3:6, :],
                   preferred_element_type=jnp.float32)
         + b2a[...])
    t = jnp.maximum(t, 0.0)
    m2 = jnp.mean(t, axis=0, keepdims=True)
    d = t - m2
    v2 = jnp.mean(d * d, axis=0, keepdims=True)
    tn = d * jax.lax.rsqrt(v2 + 1e-5) * g2[...] + be2[...]
    out[...] = jnp.dot(_bf(tn), _bf(w2b[...]),
                       preferred_element_type=jnp.float32) + b2b[...]


def kernel(pos, helix, W1a, b1a, g1, be1, W1b, b1b, W2a, b2a, g2, be2, W2b, b2b):
    n = pos.shape[0]
    nj = n // _BJ
    ni = n // _BIO
    nch = n // _CH
    perm = jnp.argsort(jnp.floor(pos[:, 0] / _W) * 256.0 + pos[:, 1])
    pos = pos[perm]
    helix = helix[perm]
    hx_c = helix.reshape(n, 1)
    hx_r = helix.reshape(nj, 1, _BJ)
    n2v = jnp.sum(pos * pos, axis=1)
    n2r = n2v.reshape(nj, 1, _BJ)
    xf = pos[:, 0].reshape(nch, _CH)
    yf = pos[:, 1].reshape(nch, _CH)
    stats_f = jnp.stack([jnp.min(xf, axis=1), jnp.max(xf, axis=1),
                         jnp.min(yf, axis=1), jnp.max(yf, axis=1),
                         jnp.max(n2v.reshape(nch, _CH), axis=1)], axis=0)
    xj = pos[:, 0].reshape(nj, _BJ)
    yj = pos[:, 1].reshape(nj, _BJ)
    stats_j = jnp.stack([jnp.min(xj, axis=1), jnp.max(xj, axis=1),
                         jnp.min(yj, axis=1), jnp.max(yj, axis=1),
                         jnp.max(n2v.reshape(nj, _BJ), axis=1)], axis=0)
    b1a2 = b1a.reshape(1, -1)

    hcol, cnt, sumh, sumh2 = pl.pallas_call(
        _phase1_body,
        grid=(nj, ni),
        in_specs=[
            pl.BlockSpec((5, nch), lambda jb, ib: (0, 0),
                         memory_space=pltpu.SMEM),
            pl.BlockSpec((5, nj), lambda jb, ib: (0, 0),
                         memory_space=pltpu.SMEM),
            pl.BlockSpec((_BIO, 3), lambda jb, ib: (ib, 0)),
            pl.BlockSpec((_BJ, 3), lambda jb, ib: (jb, 0)),
            pl.BlockSpec((_BIO, 1), lambda jb, ib: (ib, 0)),
            pl.BlockSpec((1, 1, _BJ), lambda jb, ib: (jb, 0, 0)),
            pl.BlockSpec((1, 1, _BJ), lambda jb, ib: (jb, 0, 0)),
            pl.BlockSpec((6, 128), lambda jb, ib: (0, 0)),
            pl.BlockSpec((1, 128), lambda jb, ib: (0, 0)),
        ],
        out_specs=[
            pl.BlockSpec((_BJ, 128), lambda jb, ib: (jb, 0)),
            pl.BlockSpec((1, 1, _BJ), lambda jb, ib: (jb, 0, 0)),
            pl.BlockSpec((1, 128), lambda jb, ib: (0, 0)),
            pl.BlockSpec((1, 128), lambda jb, ib: (0, 0)),
        ],
        out_shape=[
            jax.ShapeDtypeStruct((n, 128), jnp.float32),
            jax.ShapeDtypeStruct((nj, 1, _BJ), jnp.float32),
            jax.ShapeDtypeStruct((1, 128), jnp.float32),
            jax.ShapeDtypeStruct((1, 128), jnp.float32),
        ],
        compiler_params=pltpu.CompilerParams(
            dimension_semantics=("arbitrary", "arbitrary")),
    )(stats_f, stats_j, pos, pos, hx_c, hx_r, n2r, W1a, b1a2)

    out = pl.pallas_call(
        _phase2_body,
        out_shape=jax.ShapeDtypeStruct((n, 3), jnp.float32),
    )(pos, hcol, cnt.reshape(n, 1), sumh, sumh2,
      W1b, b1b.reshape(1, -1), g1.reshape(1, -1), be1.reshape(1, -1),
      W2a, b2a.reshape(1, -1), g2.reshape(1, -1), be2.reshape(1, -1),
      W2b, b2b.reshape(1, -1))
    inv = jnp.zeros((n,), jnp.int32).at[perm].set(
        jnp.arange(n, dtype=jnp.int32))
    return out[inv]
